# initial kernel scaffold (unmeasured)
import jax
import jax.numpy as jnp
from jax import lax
from jax.experimental import pallas as pl
from jax.experimental.pallas import tpu as pltpu

N_Z = 4


def kernel(ids, E):
    T = ids.shape[0]
    V_sh, D = E.shape

    def body(ids_ref, e_ref, out_ref, comm_ref, send_sems, recv_sems):
        my_x = lax.axis_index("x")
        my_y = lax.axis_index("y")
        my_z = lax.axis_index("z")
        right = (my_z + 1) % N_Z

        local_ids = ids_ref[:, :] - my_z * V_sh
        iota = lax.broadcasted_iota(jnp.int32, (T, V_sh), 1)
        onehot = (local_ids == iota).astype(jnp.bfloat16)
        partial = lax.dot_general(
            onehot,
            e_ref[:, :].astype(jnp.bfloat16),
            dimension_numbers=(((1,), (0,)), ((), ())),
            preferred_element_type=jnp.float32,
        )
        out_ref[:, :] = partial
        comm_ref[0, :, :] = partial.astype(jnp.bfloat16)

        for h in range(N_Z - 1):
            rdma = pltpu.make_async_remote_copy(
                src_ref=comm_ref.at[h],
                dst_ref=comm_ref.at[h + 1],
                send_sem=send_sems.at[h],
                recv_sem=recv_sems.at[h],
                device_id=(my_x, my_y, right),
                device_id_type=pl.DeviceIdType.MESH,
            )
            rdma.start()
            rdma.wait()
            out_ref[:, :] += comm_ref[h + 1, :, :].astype(jnp.float32)

    ids_2d = ids.reshape(T, 1)
    return pl.pallas_call(
        body,
        out_shape=jax.ShapeDtypeStruct((T, D), jnp.float32),
        in_specs=[
            pl.BlockSpec(memory_space=pltpu.VMEM),
            pl.BlockSpec(memory_space=pltpu.VMEM),
        ],
        out_specs=pl.BlockSpec(memory_space=pltpu.VMEM),
        scratch_shapes=[
            pltpu.VMEM((N_Z, T, D), jnp.bfloat16),
            pltpu.SemaphoreType.DMA((N_Z - 1,)),
            pltpu.SemaphoreType.DMA((N_Z - 1,)),
        ],
        compiler_params=pltpu.CompilerParams(collective_id=0),
    )(ids_2d, E)


# baseline (device time: 41123 ns/iter reference)
import jax
import jax.numpy as jnp
from jax import lax
from jax.experimental import pallas as pl
from jax.experimental.pallas import tpu as pltpu

N_Z = 4


def kernel(ids, E):
    T = ids.shape[0]
    V_sh, D = E.shape

    def body(ids_ref, e_ref, out_ref, comm_ref, send_sems, recv_sems):
        my_x = lax.axis_index("x")
        my_y = lax.axis_index("y")
        my_z = lax.axis_index("z")
        right = (my_z + 1) % N_Z

        local_ids = ids_ref[:, :] - my_z * V_sh
        iota = lax.broadcasted_iota(jnp.int32, (T, V_sh), 1)
        onehot = (local_ids == iota).astype(jnp.bfloat16)
        partial = lax.dot_general(
            onehot,
            e_ref[:, :].astype(jnp.bfloat16),
            dimension_numbers=(((1,), (0,)), ((), ())),
            preferred_element_type=jnp.float32,
        )
        out_ref[:, :] = partial
        comm_ref[0, :, :] = partial.astype(jnp.bfloat16)

        for h in range(N_Z - 1):
            rdma = pltpu.make_async_remote_copy(
                src_ref=comm_ref.at[h],
                dst_ref=comm_ref.at[h + 1],
                send_sem=send_sems.at[h],
                recv_sem=recv_sems.at[h],
                device_id=(my_x, my_y, right),
                device_id_type=pl.DeviceIdType.MESH,
            )
            rdma.start()
            rdma.wait()
            out_ref[:, :] += comm_ref[h + 1, :, :].astype(jnp.float32)

    ids_2d = ids.reshape(T, 1)
    return pl.pallas_call(
        body,
        out_shape=jax.ShapeDtypeStruct((T, D), jnp.float32),
        in_specs=[
            pl.BlockSpec(memory_space=pltpu.VMEM),
            pl.BlockSpec(memory_space=pltpu.VMEM),
        ],
        out_specs=pl.BlockSpec(memory_space=pltpu.VMEM),
        scratch_shapes=[
            pltpu.VMEM((N_Z, T, D), jnp.bfloat16),
            pltpu.SemaphoreType.DMA((N_Z - 1,)),
            pltpu.SemaphoreType.DMA((N_Z - 1,)),
        ],
    )(ids_2d, E)


# device time: 31150 ns/iter; 1.3202x vs baseline; 1.3202x over previous
import jax
import jax.numpy as jnp
from jax import lax
from jax.experimental import pallas as pl
from jax.experimental.pallas import tpu as pltpu

N_X, N_Y, N_Z = 2, 4, 4
N_COL = N_X * N_Y


def kernel(ids, E):
    T = ids.shape[0]
    V_sh, D = E.shape
    C = T // N_COL

    def body(ids_ref, e_ref, out_ref,
             zbuf_ref, gather_ref,
             z_send_sems, z_recv_sems,
             y_send_sems, y_recv_sems,
             x_send_sem, x_recv_sem):
        my_x = lax.axis_index("x")
        my_y = lax.axis_index("y")
        my_z = lax.axis_index("z")
        col = my_x * N_Y + my_y

        my_ids = ids_ref[pl.ds(col * C, C), :] - my_z * V_sh
        iota = lax.broadcasted_iota(jnp.int32, (C, V_sh), 1)
        onehot = (my_ids == iota).astype(jnp.float32)
        partial = lax.dot_general(
            onehot, e_ref[:, :],
            dimension_numbers=(((1,), (0,)), ((), ())),
            preferred_element_type=jnp.float32,
        )
        my_chunk = partial.astype(jnp.bfloat16)

        zbuf_ref[0, :, :] = my_chunk
        z_rdmas = []
        for k in (1, 2, 3):
            tgt_z = lax.rem(my_z + k, N_Z)
            rdma = pltpu.make_async_remote_copy(
                src_ref=zbuf_ref.at[0],
                dst_ref=zbuf_ref.at[k],
                send_sem=z_send_sems.at[k - 1],
                recv_sem=z_recv_sems.at[k - 1],
                device_id=(my_x, my_y, tgt_z),
                device_id_type=pl.DeviceIdType.MESH,
            )
            rdma.start()
            z_rdmas.append(rdma)
        for rdma in z_rdmas:
            rdma.wait()
        reduced = (zbuf_ref[0, :, :].astype(jnp.float32)
                   + zbuf_ref[1, :, :].astype(jnp.float32)
                   + zbuf_ref[2, :, :].astype(jnp.float32)
                   + zbuf_ref[3, :, :].astype(jnp.float32))
        gather_ref[col, :, :] = reduced.astype(jnp.bfloat16)

        y_rdmas = []
        for k in (1, 2, 3):
            tgt_y = lax.rem(my_y + k, N_Y)
            rdma = pltpu.make_async_remote_copy(
                src_ref=gather_ref.at[col],
                dst_ref=gather_ref.at[col],
                send_sem=y_send_sems.at[k - 1],
                recv_sem=y_recv_sems.at[k - 1],
                device_id=(my_x, tgt_y, my_z),
                device_id_type=pl.DeviceIdType.MESH,
            )
            rdma.start()
            y_rdmas.append(rdma)
        for rdma in y_rdmas:
            rdma.wait()

        plane = pl.ds(my_x * N_Y, N_Y)
        x_rdma = pltpu.make_async_remote_copy(
            src_ref=gather_ref.at[plane],
            dst_ref=gather_ref.at[plane],
            send_sem=x_send_sem,
            recv_sem=x_recv_sem,
            device_id=(1 - my_x, my_y, my_z),
            device_id_type=pl.DeviceIdType.MESH,
        )
        x_rdma.start()
        x_rdma.wait()

        out_ref[:, :] = gather_ref[:, :, :].reshape(T, D).astype(jnp.float32)

    ids_2d = ids.reshape(T, 1)
    return pl.pallas_call(
        body,
        out_shape=jax.ShapeDtypeStruct((T, D), jnp.float32),
        in_specs=[
            pl.BlockSpec(memory_space=pltpu.VMEM),
            pl.BlockSpec(memory_space=pltpu.VMEM),
        ],
        out_specs=pl.BlockSpec(memory_space=pltpu.VMEM),
        scratch_shapes=[
            pltpu.VMEM((N_Z, C, D), jnp.bfloat16),
            pltpu.VMEM((N_COL, C, D), jnp.bfloat16),
            pltpu.SemaphoreType.DMA((3,)),
            pltpu.SemaphoreType.DMA((3,)),
            pltpu.SemaphoreType.DMA((3,)),
            pltpu.SemaphoreType.DMA((3,)),
            pltpu.SemaphoreType.DMA,
            pltpu.SemaphoreType.DMA,
        ],
    )(ids_2d, E)


# device time: 29469 ns/iter; 1.3955x vs baseline; 1.0570x over previous
import jax
import jax.numpy as jnp
from jax import lax
from jax.experimental import pallas as pl
from jax.experimental.pallas import tpu as pltpu

N_X, N_Y, N_Z = 2, 4, 4
N_COL = N_X * N_Y


def kernel(ids, E):
    T = ids.shape[0]
    V_sh, D = E.shape
    C = T // N_COL

    def body(ids_ref, e_ref, out_ref,
             zbuf_ref, gather_ref,
             z_send_sems, z_recv_sems,
             xy_send_sems, xy_recv_sems):
        my_x = lax.axis_index("x")
        my_y = lax.axis_index("y")
        my_z = lax.axis_index("z")
        col = my_x * N_Y + my_y

        my_ids = ids_ref[pl.ds(col * C, C), :] - my_z * V_sh
        iota = lax.broadcasted_iota(jnp.int32, (C, V_sh), 1)
        onehot = (my_ids == iota).astype(jnp.float32)
        partial = lax.dot_general(
            onehot, e_ref[:, :],
            dimension_numbers=(((1,), (0,)), ((), ())),
            preferred_element_type=jnp.float32,
        )
        zbuf_ref[0, :, :] = partial.astype(jnp.bfloat16)

        z_rdmas = []
        for k in (1, 2, 3):
            rdma = pltpu.make_async_remote_copy(
                src_ref=zbuf_ref.at[0],
                dst_ref=zbuf_ref.at[k],
                send_sem=z_send_sems.at[k - 1],
                recv_sem=z_recv_sems.at[k - 1],
                device_id=(my_x, my_y, lax.rem(my_z + k, N_Z)),
                device_id_type=pl.DeviceIdType.MESH,
            )
            rdma.start()
            z_rdmas.append(rdma)
        for rdma in z_rdmas:
            rdma.wait()
        reduced = (zbuf_ref[0, :, :].astype(jnp.float32)
                   + zbuf_ref[1, :, :].astype(jnp.float32)
                   + zbuf_ref[2, :, :].astype(jnp.float32)
                   + zbuf_ref[3, :, :].astype(jnp.float32))
        gather_ref[col, :, :] = reduced.astype(jnp.bfloat16)

        send_rdmas = []
        for k in range(1, N_COL):
            tcol = lax.rem(col + k, N_COL)
            rdma = pltpu.make_async_remote_copy(
                src_ref=gather_ref.at[col],
                dst_ref=gather_ref.at[col],
                send_sem=xy_send_sems.at[k - 1],
                recv_sem=xy_recv_sems.at[k - 1],
                device_id=(tcol // N_Y, lax.rem(tcol, N_Y), my_z),
                device_id_type=pl.DeviceIdType.MESH,
            )
            rdma.start()
            send_rdmas.append(rdma)
        for k in range(1, N_COL):
            scol = lax.rem(col + N_COL - k, N_COL)
            recv = pltpu.make_async_remote_copy(
                src_ref=gather_ref.at[scol],
                dst_ref=gather_ref.at[scol],
                send_sem=xy_send_sems.at[0],
                recv_sem=xy_recv_sems.at[k - 1],
                device_id=(my_x, my_y, my_z),
                device_id_type=pl.DeviceIdType.MESH,
            )
            recv.wait_recv()
        for rdma in send_rdmas:
            rdma.wait_send()

        out_ref[:, :] = gather_ref[:, :, :].reshape(T, D).astype(jnp.float32)

    ids_2d = ids.reshape(T, 1)
    return pl.pallas_call(
        body,
        out_shape=jax.ShapeDtypeStruct((T, D), jnp.float32),
        in_specs=[
            pl.BlockSpec(memory_space=pltpu.VMEM),
            pl.BlockSpec(memory_space=pltpu.VMEM),
        ],
        out_specs=pl.BlockSpec(memory_space=pltpu.VMEM),
        scratch_shapes=[
            pltpu.VMEM((N_Z, C, D), jnp.bfloat16),
            pltpu.VMEM((N_COL, C, D), jnp.bfloat16),
            pltpu.SemaphoreType.DMA((3,)),
            pltpu.SemaphoreType.DMA((3,)),
            pltpu.SemaphoreType.DMA((N_COL - 1,)),
            pltpu.SemaphoreType.DMA((N_COL - 1,)),
        ],
    )(ids_2d, E)
